# baseline (speedup 1.0000x reference)
import functools
import jax
import jax.numpy as jnp
from jax import lax
from jax.experimental import pallas as pl
from jax.experimental.pallas import tpu as pltpu
from jax.experimental.pallas import tpu_sc as plsc

N_NODES = 10000
N_EDGES = 5000
NNZ = 320000
D = 128

NC = 2
NS = 16
EPAD = 5120
VPAD = 10240
E_STRIPE = EPAD // NS
V_STRIPE = VPAD // NS

BLK = 320
SUB1 = 10240
NB = SUB1 // BLK


def _mm_body(x_ref, w_ref, o_ref):
    o_ref[...] = lax.dot_general(
        x_ref[...], w_ref[...], (((1,), (1,)), ((), ())),
        preferred_element_type=jnp.float32)


def _matmul(X, W):
    grid = 10
    rb = N_NODES // grid
    return pl.pallas_call(
        _mm_body,
        grid=(grid,),
        in_specs=[
            pl.BlockSpec((rb, D), lambda i: (i, 0)),
            pl.BlockSpec((D, D), lambda i: (0, 0)),
        ],
        out_specs=pl.BlockSpec((rb, D), lambda i: (i, 0)),
        out_shape=jax.ShapeDtypeStruct((N_NODES, D), jnp.float32),
    )(X, W)


def _phase1_body(zeros_hbm, xw_hbm, vidx_hbm, hidx_hbm, xe_out_hbm,
                 vb0, vb1, hb0, hb1, rows0, rows1, xe_sh,
                 isem, gsem0, gsem1):
    c = lax.axis_index("c")
    s = lax.axis_index("s")
    pltpu.sync_copy(zeros_hbm.at[pl.ds(s * E_STRIPE, E_STRIPE)],
                    xe_sh.at[pl.ds(s * E_STRIPE, E_STRIPE)])
    base = (c * NS + s) * SUB1
    vb = (vb0, vb1)
    hb = (hb0, hb1)
    rows = (rows0, rows1)
    isem = (isem, isem)
    gsem = (gsem0, gsem1)
    iv = [None, None]
    ih = [None, None]
    gd = [None, None]
    for q in (0, 1):
        off = base + q * BLK
        iv[q] = pltpu.async_copy(vidx_hbm.at[pl.ds(off, BLK)], vb[q], isem[q])
        ih[q] = pltpu.async_copy(hidx_hbm.at[pl.ds(off, BLK)], hb[q], isem[q])
    plsc.subcore_barrier()
    iv[0].wait()
    ih[0].wait()
    gd[0] = pltpu.async_copy(xw_hbm.at[vb[0]], rows[0], gsem[0])
    for b in range(NB):
        p = b % 2
        if b + 1 < NB:
            iv[1 - p].wait()
            ih[1 - p].wait()
            gd[1 - p] = pltpu.async_copy(
                xw_hbm.at[vb[1 - p]], rows[1 - p], gsem[1 - p])
        gd[p].wait()
        pltpu.sync_copy(rows[p], xe_sh.at[hb[p]], add=True)
        if b + 2 < NB:
            off = base + (b + 2) * BLK
            iv[p] = pltpu.async_copy(
                vidx_hbm.at[pl.ds(off, BLK)], vb[p], isem[p])
            ih[p] = pltpu.async_copy(
                hidx_hbm.at[pl.ds(off, BLK)], hb[p], isem[p])
    plsc.subcore_barrier()
    pltpu.sync_copy(xe_sh.at[pl.ds(s * E_STRIPE, E_STRIPE)],
                    xe_out_hbm.at[c, pl.ds(s * E_STRIPE, E_STRIPE)])


def _phase1(zeros, Xw, vidx, hidx):
    mesh = plsc.VectorSubcoreMesh(core_axis_name="c", subcore_axis_name="s")
    k = functools.partial(
        pl.kernel,
        out_type=jax.ShapeDtypeStruct((NC, EPAD, D), jnp.float32),
        mesh=mesh,
        scratch_types=[
            pltpu.VMEM((BLK,), jnp.int32),
            pltpu.VMEM((BLK,), jnp.int32),
            pltpu.VMEM((BLK,), jnp.int32),
            pltpu.VMEM((BLK,), jnp.int32),
            pltpu.VMEM((BLK, D), jnp.float32),
            pltpu.VMEM((BLK, D), jnp.float32),
            pltpu.VMEM_SHARED((EPAD, D), jnp.float32),
            pltpu.SemaphoreType.DMA,
            pltpu.SemaphoreType.DMA,
            pltpu.SemaphoreType.DMA,
        ],
    )(_phase1_body)
    return k(zeros, Xw, vidx, hidx)


def _edge_body(xe0_ref, xe1_ref, dege_ref, o_ref):
    o_ref[...] = (xe0_ref[...] + xe1_ref[...]) / dege_ref[...]


def _edge_scale(xe_part, degE_pad):
    grid = 8
    rb = EPAD // grid
    return pl.pallas_call(
        _edge_body,
        grid=(grid,),
        in_specs=[
            pl.BlockSpec((rb, D), lambda i: (i, 0)),
            pl.BlockSpec((rb, D), lambda i: (i, 0)),
            pl.BlockSpec((rb, 1), lambda i: (i, 0)),
        ],
        out_specs=pl.BlockSpec((rb, D), lambda i: (i, 0)),
        out_shape=jax.ShapeDtypeStruct((EPAD, D), jnp.float32),
    )(xe_part[0], xe_part[1], degE_pad.reshape(EPAD, 1))


VHALF = N_NODES // NC
VACC = 5248
V_STRIPE2 = VACC // NS
SUB2 = 20480
NB2 = SUB2 // BLK


def _phase2_body(zeros_hbm, xe_hbm, vidx_hbm, hidx_hbm, xv_out_hbm,
                 vb0, vb1, hb0, hb1, sb0, sb1, rows0, rows1, xv_sh,
                 isem, gsem0, gsem1):
    c = lax.axis_index("c")
    s = lax.axis_index("s")
    pltpu.sync_copy(zeros_hbm.at[pl.ds(s * V_STRIPE2, V_STRIPE2)],
                    xv_sh.at[pl.ds(s * V_STRIPE2, V_STRIPE2)])
    vbase = c * VHALF
    base = s * SUB2
    vb = (vb0, vb1)
    hb = (hb0, hb1)
    sb = (sb0, sb1)
    rows = (rows0, rows1)
    isem = (isem, isem)
    gsem = (gsem0, gsem1)
    iv = [None, None]
    ih = [None, None]
    gd = [None, None]

    def ridx_blk(q):
        def ridx(i, _):
            sl = pl.ds(i * 16, 16)
            t = vb[q][sl] - vbase
            ok = (t >= 0) & (t < VHALF)
            trash = VHALF + (vb[q][sl] & 127)
            sb[q][sl] = jnp.where(ok, t, trash)
            return 0

        lax.fori_loop(0, BLK // 16, ridx, 0)

    for q in (0, 1):
        off = base + q * BLK
        iv[q] = pltpu.async_copy(vidx_hbm.at[pl.ds(off, BLK)], vb[q], isem[q])
        ih[q] = pltpu.async_copy(hidx_hbm.at[pl.ds(off, BLK)], hb[q], isem[q])
    plsc.subcore_barrier()
    iv[0].wait()
    ih[0].wait()
    gd[0] = pltpu.async_copy(xe_hbm.at[hb[0]], rows[0], gsem[0])
    ridx_blk(0)
    for b in range(NB2):
        p = b % 2
        if b + 1 < NB2:
            iv[1 - p].wait()
            ih[1 - p].wait()
            gd[1 - p] = pltpu.async_copy(
                xe_hbm.at[hb[1 - p]], rows[1 - p], gsem[1 - p])
            ridx_blk(1 - p)
        gd[p].wait()
        pltpu.sync_copy(rows[p], xv_sh.at[sb[p]], add=True)
        if b + 2 < NB2:
            off = base + (b + 2) * BLK
            iv[p] = pltpu.async_copy(
                vidx_hbm.at[pl.ds(off, BLK)], vb[p], isem[p])
            ih[p] = pltpu.async_copy(
                hidx_hbm.at[pl.ds(off, BLK)], hb[p], isem[p])
    plsc.subcore_barrier()
    @pl.when(s < NS - 1)
    def _():
        pltpu.sync_copy(
            xv_sh.at[pl.ds(s * 312, 312)],
            xv_out_hbm.at[pl.ds(vbase + s * 312, 312)])

    @pl.when(s == NS - 1)
    def _():
        pltpu.sync_copy(
            xv_sh.at[pl.ds(4680, 320)],
            xv_out_hbm.at[pl.ds(vbase + 4680, 320)])


def _phase2(zeros, xe, vidx, hidx):
    mesh = plsc.VectorSubcoreMesh(core_axis_name="c", subcore_axis_name="s")
    k = functools.partial(
        pl.kernel,
        out_type=jax.ShapeDtypeStruct((VPAD, D), jnp.float32),
        mesh=mesh,
        scratch_types=[
            pltpu.VMEM((BLK,), jnp.int32),
            pltpu.VMEM((BLK,), jnp.int32),
            pltpu.VMEM((BLK,), jnp.int32),
            pltpu.VMEM((BLK,), jnp.int32),
            pltpu.VMEM((BLK,), jnp.int32),
            pltpu.VMEM((BLK,), jnp.int32),
            pltpu.VMEM((BLK, D), jnp.float32),
            pltpu.VMEM((BLK, D), jnp.float32),
            pltpu.VMEM_SHARED((VACC, D), jnp.float32),
            pltpu.SemaphoreType.DMA,
            pltpu.SemaphoreType.DMA,
            pltpu.SemaphoreType.DMA,
        ],
    )(_phase2_body)
    return k(zeros, xe, vidx, hidx)


def _comb_body(eps_ref, xw_ref, degv_ref, xv_ref, o_ref):
    o_ref[...] = ((1.0 + eps_ref[0, 0]) * xw_ref[...]
                  + degv_ref[...] * xv_ref[...])


def _combine(eps, Xw, degV, xv):
    grid = 10
    rb = N_NODES // grid
    return pl.pallas_call(
        _comb_body,
        grid=(grid,),
        in_specs=[
            pl.BlockSpec((1, 1), lambda i: (0, 0)),
            pl.BlockSpec((rb, D), lambda i: (i, 0)),
            pl.BlockSpec((rb, 1), lambda i: (i, 0)),
            pl.BlockSpec((rb, D), lambda i: (i, 0)),
        ],
        out_specs=pl.BlockSpec((rb, D), lambda i: (i, 0)),
        out_shape=jax.ShapeDtypeStruct((N_NODES, D), jnp.float32),
    )(eps.reshape(1, 1), Xw, degV.reshape(N_NODES, 1), xv)


@jax.jit
def kernel(X, W, eps, degE, degV, vertex_index, hyperedge_index):
    perm = jnp.arange(NNZ).reshape(NNZ // 128, 128).T.reshape(NNZ)
    vidx = vertex_index.astype(jnp.int32)[perm]
    hidx = hyperedge_index.astype(jnp.int32)[perm]
    n1 = NNZ // (NC * NS)
    v1 = jnp.concatenate(
        [vidx.reshape(NC * NS, n1),
         jnp.zeros((NC * NS, SUB1 - n1), jnp.int32)], axis=1).reshape(-1)
    h1 = jnp.concatenate(
        [hidx.reshape(NC * NS, n1),
         jnp.full((NC * NS, SUB1 - n1), N_EDGES + 100, jnp.int32)],
        axis=1).reshape(-1)
    n2 = NNZ // NS
    v2 = jnp.concatenate(
        [vidx.reshape(NS, n2),
         jnp.full((NS, SUB2 - n2), N_NODES, jnp.int32)], axis=1).reshape(-1)
    h2 = jnp.concatenate(
        [hidx.reshape(NS, n2),
         jnp.zeros((NS, SUB2 - n2), jnp.int32)], axis=1).reshape(-1)
    zeros = jnp.zeros((VPAD, D), jnp.float32)
    degE_pad = jnp.concatenate(
        [degE, jnp.ones((EPAD - N_EDGES,), jnp.float32)])

    Xw = _matmul(X, W)
    xe_part = _phase1(zeros, Xw, v1, h1)
    xe = _edge_scale(xe_part, degE_pad)
    xv = _phase2(zeros, xe, v2, h2)
    out = _combine(eps, Xw, degV, xv[:N_NODES])
    return out


# speedup vs baseline: 2.4923x; 2.4923x over previous
import functools
import jax
import jax.numpy as jnp
from jax import lax
from jax.experimental import pallas as pl
from jax.experimental.pallas import tpu as pltpu
from jax.experimental.pallas import tpu_sc as plsc

N_NODES = 10000
N_EDGES = 5000
NNZ = 320000
D = 128

NC = 2
NS = 16
EPAD = 5120
VPAD = 10240
E_STRIPE = EPAD // NS
V_STRIPE = VPAD // NS

NNZ_PER_SUB = NNZ // (NC * NS)
BLK = 400
NB = NNZ_PER_SUB // BLK


def _mm_body(x_ref, w_ref, o_ref):
    o_ref[...] = lax.dot_general(
        x_ref[...], w_ref[...], (((1,), (1,)), ((), ())),
        preferred_element_type=jnp.float32)


def _matmul(X, W):
    grid = 10
    rb = N_NODES // grid
    return pl.pallas_call(
        _mm_body,
        grid=(grid,),
        in_specs=[
            pl.BlockSpec((rb, D), lambda i: (i, 0)),
            pl.BlockSpec((D, D), lambda i: (0, 0)),
        ],
        out_specs=pl.BlockSpec((rb, D), lambda i: (i, 0)),
        out_shape=jax.ShapeDtypeStruct((N_NODES, D), jnp.float32),
    )(X, W)


def _phase1_body(zeros_hbm, xw_hbm, vidx_hbm, hidx_hbm, xe_out_hbm,
                 vb0, vb1, hb0, hb1, rows_v, xe_sh, isem, gsem):
    c = lax.axis_index("c")
    s = lax.axis_index("s")
    pltpu.sync_copy(zeros_hbm.at[pl.ds(s * E_STRIPE, E_STRIPE)],
                    xe_sh.at[pl.ds(s * E_STRIPE, E_STRIPE)])
    base = (c * NS + s) * NNZ_PER_SUB
    vb = (vb0, vb1)
    hb = (hb0, hb1)
    iv = [None, None]
    ih = [None, None]
    iv[0] = pltpu.async_copy(vidx_hbm.at[pl.ds(base, BLK)], vb[0], isem)
    ih[0] = pltpu.async_copy(hidx_hbm.at[pl.ds(base, BLK)], hb[0], isem)
    plsc.subcore_barrier()
    for b in range(NB):
        p = b % 2
        iv[p].wait()
        ih[p].wait()
        gd = pltpu.async_copy(xw_hbm.at[vb[p]], rows_v, gsem)
        if b + 1 < NB:
            off = base + (b + 1) * BLK
            iv[1 - p] = pltpu.async_copy(
                vidx_hbm.at[pl.ds(off, BLK)], vb[1 - p], isem)
            ih[1 - p] = pltpu.async_copy(
                hidx_hbm.at[pl.ds(off, BLK)], hb[1 - p], isem)
        gd.wait()
        pltpu.sync_copy(rows_v, xe_sh.at[hb[p]], add=True)
    plsc.subcore_barrier()
    pltpu.sync_copy(xe_sh.at[pl.ds(s * E_STRIPE, E_STRIPE)],
                    xe_out_hbm.at[c, pl.ds(s * E_STRIPE, E_STRIPE)])


def _phase1(zeros, Xw, vidx, hidx):
    mesh = plsc.VectorSubcoreMesh(core_axis_name="c", subcore_axis_name="s")
    k = functools.partial(
        pl.kernel,
        out_type=jax.ShapeDtypeStruct((NC, EPAD, D), jnp.float32),
        mesh=mesh,
        scratch_types=[
            pltpu.VMEM((BLK,), jnp.int32),
            pltpu.VMEM((BLK,), jnp.int32),
            pltpu.VMEM((BLK,), jnp.int32),
            pltpu.VMEM((BLK,), jnp.int32),
            pltpu.VMEM((BLK, D), jnp.float32),
            pltpu.VMEM_SHARED((EPAD, D), jnp.float32),
            pltpu.SemaphoreType.DMA,
            pltpu.SemaphoreType.DMA,
        ],
    )(_phase1_body)
    return k(zeros, Xw, vidx, hidx)


def _edge_body(xe0_ref, xe1_ref, dege_ref, o_ref):
    o_ref[...] = (xe0_ref[...] + xe1_ref[...]) / dege_ref[...]


def _edge_scale(xe_part, degE_pad):
    grid = 8
    rb = EPAD // grid
    return pl.pallas_call(
        _edge_body,
        grid=(grid,),
        in_specs=[
            pl.BlockSpec((rb, D), lambda i: (i, 0)),
            pl.BlockSpec((rb, D), lambda i: (i, 0)),
            pl.BlockSpec((rb, 1), lambda i: (i, 0)),
        ],
        out_specs=pl.BlockSpec((rb, D), lambda i: (i, 0)),
        out_shape=jax.ShapeDtypeStruct((EPAD, D), jnp.float32),
    )(xe_part[0], xe_part[1], degE_pad.reshape(EPAD, 1))


VHALF = N_NODES // NC
VACC = 5248
V_STRIPE2 = VACC // NS
NNZ_PER_SUB2 = NNZ // NS
NB2 = NNZ_PER_SUB2 // BLK


def _phase2_body(zeros_hbm, xe_hbm, vidx_hbm, hidx_hbm, xv_out_hbm,
                 vb0, vb1, hb0, hb1, sb0, sb1, rows_v, xv_sh, isem, gsem):
    c = lax.axis_index("c")
    s = lax.axis_index("s")
    pltpu.sync_copy(zeros_hbm.at[pl.ds(s * V_STRIPE2, V_STRIPE2)],
                    xv_sh.at[pl.ds(s * V_STRIPE2, V_STRIPE2)])
    vbase = c * VHALF
    base = s * NNZ_PER_SUB2
    vb = (vb0, vb1)
    hb = (hb0, hb1)
    sb = (sb0, sb1)
    iv = [None, None]
    ih = [None, None]
    iv[0] = pltpu.async_copy(vidx_hbm.at[pl.ds(base, BLK)], vb[0], isem)
    ih[0] = pltpu.async_copy(hidx_hbm.at[pl.ds(base, BLK)], hb[0], isem)
    plsc.subcore_barrier()
    for b in range(NB2):
        p = b % 2
        iv[p].wait()
        ih[p].wait()
        gd = pltpu.async_copy(xe_hbm.at[hb[p]], rows_v, gsem)
        if b + 1 < NB2:
            off = base + (b + 1) * BLK
            iv[1 - p] = pltpu.async_copy(
                vidx_hbm.at[pl.ds(off, BLK)], vb[1 - p], isem)
            ih[1 - p] = pltpu.async_copy(
                hidx_hbm.at[pl.ds(off, BLK)], hb[1 - p], isem)

        def ridx(i, _, p=p):
            sl = pl.ds(i * 16, 16)
            t = vb[p][sl] - vbase
            ok = (t >= 0) & (t < VHALF)
            trash = VHALF + (vb[p][sl] & 127)
            sb[p][sl] = jnp.where(ok, t, trash)
            return 0

        lax.fori_loop(0, BLK // 16, ridx, 0)
        gd.wait()
        pltpu.sync_copy(rows_v, xv_sh.at[sb[p]], add=True)
    plsc.subcore_barrier()
    @pl.when(s < NS - 1)
    def _():
        pltpu.sync_copy(
            xv_sh.at[pl.ds(s * 312, 312)],
            xv_out_hbm.at[pl.ds(vbase + s * 312, 312)])

    @pl.when(s == NS - 1)
    def _():
        pltpu.sync_copy(
            xv_sh.at[pl.ds(4680, 320)],
            xv_out_hbm.at[pl.ds(vbase + 4680, 320)])


def _phase2(zeros, xe, vidx, hidx):
    mesh = plsc.VectorSubcoreMesh(core_axis_name="c", subcore_axis_name="s")
    k = functools.partial(
        pl.kernel,
        out_type=jax.ShapeDtypeStruct((VPAD, D), jnp.float32),
        mesh=mesh,
        scratch_types=[
            pltpu.VMEM((BLK,), jnp.int32),
            pltpu.VMEM((BLK,), jnp.int32),
            pltpu.VMEM((BLK,), jnp.int32),
            pltpu.VMEM((BLK,), jnp.int32),
            pltpu.VMEM((BLK,), jnp.int32),
            pltpu.VMEM((BLK,), jnp.int32),
            pltpu.VMEM((BLK, D), jnp.float32),
            pltpu.VMEM_SHARED((VACC, D), jnp.float32),
            pltpu.SemaphoreType.DMA,
            pltpu.SemaphoreType.DMA,
        ],
    )(_phase2_body)
    return k(zeros, xe, vidx, hidx)


def _comb_body(eps_ref, xw_ref, degv_ref, xv_ref, o_ref):
    o_ref[...] = ((1.0 + eps_ref[0, 0]) * xw_ref[...]
                  + degv_ref[...] * xv_ref[...])


def _combine(eps, Xw, degV, xv):
    grid = 10
    rb = N_NODES // grid
    return pl.pallas_call(
        _comb_body,
        grid=(grid,),
        in_specs=[
            pl.BlockSpec((1, 1), lambda i: (0, 0)),
            pl.BlockSpec((rb, D), lambda i: (i, 0)),
            pl.BlockSpec((rb, 1), lambda i: (i, 0)),
            pl.BlockSpec((rb, D), lambda i: (i, 0)),
        ],
        out_specs=pl.BlockSpec((rb, D), lambda i: (i, 0)),
        out_shape=jax.ShapeDtypeStruct((N_NODES, D), jnp.float32),
    )(eps.reshape(1, 1), Xw, degV.reshape(N_NODES, 1), xv)


@jax.jit
def kernel(X, W, eps, degE, degV, vertex_index, hyperedge_index):
    perm = jnp.arange(NNZ).reshape(NNZ // 128, 128).T.reshape(NNZ)
    vidx = vertex_index.astype(jnp.int32)[perm]
    hidx = hyperedge_index.astype(jnp.int32)[perm]
    zeros = jnp.zeros((VPAD, D), jnp.float32)
    degE_pad = jnp.concatenate(
        [degE, jnp.ones((EPAD - N_EDGES,), jnp.float32)])

    Xw = _matmul(X, W)
    xe_part = _phase1(zeros, Xw, vidx, hidx)
    xe = _edge_scale(xe_part, degE_pad)
    xv = _phase2(zeros, xe, vidx, hidx)
    out = _combine(eps, Xw, degV, xv[:N_NODES])
    return out


# speedup vs baseline: 2.8196x; 1.1313x over previous
import functools
import jax
import jax.numpy as jnp
from jax import lax
from jax.experimental import pallas as pl
from jax.experimental.pallas import tpu as pltpu
from jax.experimental.pallas import tpu_sc as plsc

N_NODES = 10000
N_EDGES = 5000
NNZ = 320000
D = 128

NC = 2
NS = 16
EPAD = 5120
VPAD = 10240
E_STRIPE = EPAD // NS
V_STRIPE = VPAD // NS

NNZ_PER_SUB = NNZ // (NC * NS)
BLK = 400
NB = NNZ_PER_SUB // BLK


def _mm_body(x_ref, w_ref, o_ref):
    o_ref[...] = lax.dot_general(
        x_ref[...], w_ref[...], (((1,), (1,)), ((), ())),
        preferred_element_type=jnp.float32)


def _matmul(X, W):
    grid = 10
    rb = N_NODES // grid
    return pl.pallas_call(
        _mm_body,
        grid=(grid,),
        in_specs=[
            pl.BlockSpec((rb, D), lambda i: (i, 0)),
            pl.BlockSpec((D, D), lambda i: (0, 0)),
        ],
        out_specs=pl.BlockSpec((rb, D), lambda i: (i, 0)),
        out_shape=jax.ShapeDtypeStruct((N_NODES, D), jnp.float32),
    )(X, W)


def _phase1_body(zeros_hbm, xw_hbm, vidx_hbm, hidx_hbm, xe_out_hbm,
                 vb0, vb1, hb0, hb1, rows_v, xe_sh, isem, gsem):
    c = lax.axis_index("c")
    s = lax.axis_index("s")
    pltpu.sync_copy(zeros_hbm.at[pl.ds(s * E_STRIPE, E_STRIPE)],
                    xe_sh.at[pl.ds(s * E_STRIPE, E_STRIPE)])
    base = (c * NS + s) * NNZ_PER_SUB
    vb = (vb0, vb1)
    hb = (hb0, hb1)
    iv = [None, None]
    ih = [None, None]
    iv[0] = pltpu.async_copy(vidx_hbm.at[pl.ds(base, BLK)], vb[0], isem)
    ih[0] = pltpu.async_copy(hidx_hbm.at[pl.ds(base, BLK)], hb[0], isem)
    plsc.subcore_barrier()
    for b in range(NB):
        p = b % 2
        iv[p].wait()
        ih[p].wait()
        gd = pltpu.async_copy(xw_hbm.at[vb[p]], rows_v, gsem)
        if b + 1 < NB:
            off = base + (b + 1) * BLK
            iv[1 - p] = pltpu.async_copy(
                vidx_hbm.at[pl.ds(off, BLK)], vb[1 - p], isem)
            ih[1 - p] = pltpu.async_copy(
                hidx_hbm.at[pl.ds(off, BLK)], hb[1 - p], isem)
        gd.wait()
        pltpu.sync_copy(rows_v, xe_sh.at[hb[p]], add=True)
    plsc.subcore_barrier()
    pltpu.sync_copy(xe_sh.at[pl.ds(s * E_STRIPE, E_STRIPE)],
                    xe_out_hbm.at[c, pl.ds(s * E_STRIPE, E_STRIPE)])


def _phase1(zeros, Xw, vidx, hidx):
    mesh = plsc.VectorSubcoreMesh(core_axis_name="c", subcore_axis_name="s")
    k = functools.partial(
        pl.kernel,
        out_type=jax.ShapeDtypeStruct((NC, EPAD, D), jnp.float32),
        mesh=mesh,
        scratch_types=[
            pltpu.VMEM((BLK,), jnp.int32),
            pltpu.VMEM((BLK,), jnp.int32),
            pltpu.VMEM((BLK,), jnp.int32),
            pltpu.VMEM((BLK,), jnp.int32),
            pltpu.VMEM((BLK, D), jnp.float32),
            pltpu.VMEM_SHARED((EPAD, D), jnp.float32),
            pltpu.SemaphoreType.DMA,
            pltpu.SemaphoreType.DMA,
        ],
    )(_phase1_body)
    return k(zeros, Xw, vidx, hidx)


def _edge_body(xe0_ref, xe1_ref, dege_ref, o_ref):
    o_ref[...] = (xe0_ref[...] + xe1_ref[...]) / dege_ref[...]


def _edge_scale(xe_part, degE_pad):
    grid = 8
    rb = EPAD // grid
    return pl.pallas_call(
        _edge_body,
        grid=(grid,),
        in_specs=[
            pl.BlockSpec((rb, D), lambda i: (i, 0)),
            pl.BlockSpec((rb, D), lambda i: (i, 0)),
            pl.BlockSpec((rb, 1), lambda i: (i, 0)),
        ],
        out_specs=pl.BlockSpec((rb, D), lambda i: (i, 0)),
        out_shape=jax.ShapeDtypeStruct((EPAD, D), jnp.float32),
    )(xe_part[0], xe_part[1], degE_pad.reshape(EPAD, 1))


VHALF = N_NODES // NC
VACC = 5248
V_STRIPE2 = VACC // NS
NNZ_PER_SUB2 = NNZ // NS
NB2 = NNZ_PER_SUB2 // BLK


def _phase2_body(zeros_hbm, xe_hbm, vidx_hbm, hidx_hbm, xv_out_hbm,
                 vb0, vb1, hb0, hb1, sb0, sb1, rows_v, xv_sh, isem, gsem):
    c = lax.axis_index("c")
    s = lax.axis_index("s")
    pltpu.sync_copy(zeros_hbm.at[pl.ds(s * V_STRIPE2, V_STRIPE2)],
                    xv_sh.at[pl.ds(s * V_STRIPE2, V_STRIPE2)])
    vbase = c * VHALF
    base = s * NNZ_PER_SUB2
    vb = (vb0, vb1)
    hb = (hb0, hb1)
    sb = (sb0, sb1)
    iv = [None, None]
    ih = [None, None]
    iv[0] = pltpu.async_copy(vidx_hbm.at[pl.ds(base, BLK)], vb[0], isem)
    ih[0] = pltpu.async_copy(hidx_hbm.at[pl.ds(base, BLK)], hb[0], isem)
    plsc.subcore_barrier()
    for b in range(NB2):
        p = b % 2
        iv[p].wait()
        ih[p].wait()
        gd = pltpu.async_copy(xe_hbm.at[hb[p]], rows_v, gsem)
        if b + 1 < NB2:
            off = base + (b + 1) * BLK
            iv[1 - p] = pltpu.async_copy(
                vidx_hbm.at[pl.ds(off, BLK)], vb[1 - p], isem)
            ih[1 - p] = pltpu.async_copy(
                hidx_hbm.at[pl.ds(off, BLK)], hb[1 - p], isem)

        def ridx(i, _, p=p):
            sl = pl.ds(i * 16, 16)
            t = vb[p][sl] - vbase
            ok = (t >= 0) & (t < VHALF)
            trash = VHALF + (vb[p][sl] & 127)
            sb[p][sl] = jnp.where(ok, t, trash)
            return 0

        lax.fori_loop(0, BLK // 16, ridx, 0)
        gd.wait()
        pltpu.sync_copy(rows_v, xv_sh.at[sb[p]], add=True)
    plsc.subcore_barrier()
    @pl.when(s < NS - 1)
    def _():
        pltpu.sync_copy(
            xv_sh.at[pl.ds(s * 312, 312)],
            xv_out_hbm.at[pl.ds(vbase + s * 312, 312)])

    @pl.when(s == NS - 1)
    def _():
        pltpu.sync_copy(
            xv_sh.at[pl.ds(4680, 320)],
            xv_out_hbm.at[pl.ds(vbase + 4680, 320)])


def _phase2(zeros, xe, vidx, hidx):
    mesh = plsc.VectorSubcoreMesh(core_axis_name="c", subcore_axis_name="s")
    k = functools.partial(
        pl.kernel,
        out_type=jax.ShapeDtypeStruct((VPAD, D), jnp.float32),
        mesh=mesh,
        scratch_types=[
            pltpu.VMEM((BLK,), jnp.int32),
            pltpu.VMEM((BLK,), jnp.int32),
            pltpu.VMEM((BLK,), jnp.int32),
            pltpu.VMEM((BLK,), jnp.int32),
            pltpu.VMEM((BLK,), jnp.int32),
            pltpu.VMEM((BLK,), jnp.int32),
            pltpu.VMEM((BLK, D), jnp.float32),
            pltpu.VMEM_SHARED((VACC, D), jnp.float32),
            pltpu.SemaphoreType.DMA,
            pltpu.SemaphoreType.DMA,
        ],
    )(_phase2_body)
    return k(zeros, xe, vidx, hidx)


def _comb_body(eps_ref, xw_ref, degv_ref, xv_ref, o_ref):
    o_ref[...] = ((1.0 + eps_ref[0, 0]) * xw_ref[...]
                  + degv_ref[...] * xv_ref[...])


def _combine(eps, Xw, degV, xv):
    grid = 10
    rb = N_NODES // grid
    return pl.pallas_call(
        _comb_body,
        grid=(grid,),
        in_specs=[
            pl.BlockSpec((1, 1), lambda i: (0, 0)),
            pl.BlockSpec((rb, D), lambda i: (i, 0)),
            pl.BlockSpec((rb, 1), lambda i: (i, 0)),
            pl.BlockSpec((rb, D), lambda i: (i, 0)),
        ],
        out_specs=pl.BlockSpec((rb, D), lambda i: (i, 0)),
        out_shape=jax.ShapeDtypeStruct((N_NODES, D), jnp.float32),
    )(eps.reshape(1, 1), Xw, degV.reshape(N_NODES, 1), xv)


@jax.jit
def kernel(X, W, eps, degE, degV, vertex_index, hyperedge_index):
    vidx = vertex_index.astype(jnp.int32).reshape(NNZ // 128, 128).T.reshape(NNZ)
    hidx = hyperedge_index.astype(jnp.int32).reshape(NNZ // 128, 128).T.reshape(NNZ)
    zeros = jnp.zeros((VPAD, D), jnp.float32)
    degE_pad = jnp.concatenate(
        [degE, jnp.ones((EPAD - N_EDGES,), jnp.float32)])

    Xw = _matmul(X, W)
    xe_part = _phase1(zeros, Xw, vidx, hidx)
    xe = _edge_scale(xe_part, degE_pad)
    xv = _phase2(zeros, xe, vidx, hidx)
    out = _combine(eps, Xw, degV, xv[:N_NODES])
    return out
